# Initial kernel scaffold; baseline (speedup 1.0000x reference)
import jax
import jax.numpy as jnp
from jax.experimental import pallas as pl


def kernel(x, Wg, W1, W2):
    raise NotImplementedError("write your pallas kernel here")


# speedup vs baseline: 1.2492x; 1.2492x over previous
import functools

import jax
import jax.numpy as jnp
from jax.experimental import pallas as pl
from jax.experimental.pallas import tpu as pltpu

NE = 8
D_IN = 1024
D_HID = 512
CVLOSS_W = 0.01
SWITCHLOSS_W = 0.1
ZLOSS_W = 0.0001
BT = 256
N_TOK = 2048


def _moe_body(x_ref, wgt_ref, w1_ref, w2_ref, y_ref, loss_ref,
              gates_ref, gsum_ref, psum_ref, cnt_ref, zsum_ref):
    tb = pl.program_id(0)
    e = pl.program_id(1)
    nt = pl.num_programs(0)

    @pl.when(jnp.logical_and(tb == 0, e == 0))
    def _init():
        gsum_ref[...] = jnp.zeros_like(gsum_ref)
        psum_ref[...] = jnp.zeros_like(psum_ref)
        cnt_ref[...] = jnp.zeros_like(cnt_ref)
        zsum_ref[...] = jnp.zeros_like(zsum_ref)

    @pl.when(e == 0)
    def _gating():
        xb = x_ref[...]
        logits = jnp.dot(xb, wgt_ref[...])
        mx = jnp.max(logits, axis=1, keepdims=True)
        ex = jnp.exp(logits - mx)
        se = jnp.sum(ex, axis=1, keepdims=True)
        probs = ex / se
        m1 = jnp.max(probs, axis=1, keepdims=True)
        lane = jax.lax.broadcasted_iota(jnp.int32, probs.shape, 1)
        first_m1 = jnp.min(jnp.where(probs == m1, lane, NE), axis=1,
                           keepdims=True)
        probs_wo = jnp.where(lane == first_m1, -jnp.inf, probs)
        m2 = jnp.max(probs_wo, axis=1, keepdims=True)
        keep = probs >= m2
        gates = jnp.where(keep, probs, 0.0)
        gates_ref[...] = gates
        gsum_ref[...] += jnp.sum(gates, axis=0, keepdims=True)
        psum_ref[...] += jnp.sum(probs, axis=0, keepdims=True)
        cnt_ref[...] += jnp.sum(
            jnp.where(gates > 0, 1.0, 0.0), axis=0, keepdims=True)
        lse = mx[:, 0] + jnp.log(se[:, 0])
        zsum_ref[...] += jnp.sum(lse * lse).reshape(1, 1)

    h = jnp.maximum(jnp.dot(x_ref[...], w1_ref[0]), 0.0)
    o = jnp.dot(h, w2_ref[0])
    lane8 = jax.lax.broadcasted_iota(jnp.int32, (BT, NE), 1)
    ge = jnp.sum(jnp.where(lane8 == e, gates_ref[...], 0.0), axis=1,
                 keepdims=True)

    @pl.when(e == 0)
    def _y_first():
        y_ref[...] = ge * o

    @pl.when(e > 0)
    def _y_acc():
        y_ref[...] += ge * o

    @pl.when(jnp.logical_and(tb == nt - 1, e == NE - 1))
    def _loss():
        gs = gsum_ref[...]
        w = gs / jnp.maximum(jnp.sum(jnp.abs(gs)), 1e-12)
        wm = jnp.mean(w)
        var = jnp.sum((w - wm) ** 2) / (NE - 1)
        cvloss = CVLOSS_W * var / (wm * wm + 1e-10)
        pn = psum_ref[...]
        pn = pn / jnp.maximum(jnp.sum(jnp.abs(pn)), 1e-12)
        cn = cnt_ref[...]
        cn = cn / jnp.maximum(jnp.sum(jnp.abs(cn)), 1e-12)
        switchloss = SWITCHLOSS_W * (1.0 - jnp.sum(pn * cn)) * NE
        zloss = ZLOSS_W * jnp.sum(zsum_ref[...]) / N_TOK
        loss_ref[...] = (cvloss + switchloss + zloss).reshape(1, 1)


@jax.jit
def _moe_fused(xf, wgt, w1, w2):
    nt = N_TOK // BT
    y, loss = pl.pallas_call(
        _moe_body,
        grid=(nt, NE),
        in_specs=[
            pl.BlockSpec((BT, D_IN), lambda tb, e: (tb, 0)),
            pl.BlockSpec((D_IN, NE), lambda tb, e: (0, 0)),
            pl.BlockSpec((1, D_IN, D_HID), lambda tb, e: (e, 0, 0)),
            pl.BlockSpec((1, D_HID, D_IN), lambda tb, e: (e, 0, 0)),
        ],
        out_specs=[
            pl.BlockSpec((BT, D_IN), lambda tb, e: (tb, 0)),
            pl.BlockSpec((1, 1), lambda tb, e: (0, 0)),
        ],
        out_shape=[
            jax.ShapeDtypeStruct((N_TOK, D_IN), jnp.float32),
            jax.ShapeDtypeStruct((1, 1), jnp.float32),
        ],
        scratch_shapes=[
            pltpu.VMEM((BT, NE), jnp.float32),
            pltpu.VMEM((1, NE), jnp.float32),
            pltpu.VMEM((1, NE), jnp.float32),
            pltpu.VMEM((1, NE), jnp.float32),
            pltpu.VMEM((1, 1), jnp.float32),
        ],
        compiler_params=pltpu.CompilerParams(
            dimension_semantics=("arbitrary", "arbitrary"),
        ),
    )(xf, wgt, w1, w2)
    return y, loss


def kernel(x, Wg, W1, W2):
    bsz, length, emb = x.shape
    xf = x.reshape(-1, emb)
    y, loss = _moe_fused(xf, Wg.T, W1, W2)
    return y.reshape(bsz, length, emb), loss[0, 0]


# speedup vs baseline: 1.2963x; 1.0377x over previous
import functools

import jax
import jax.numpy as jnp
from jax.experimental import pallas as pl
from jax.experimental.pallas import tpu as pltpu

NE = 8
D_IN = 1024
D_HID = 512
CVLOSS_W = 0.01
SWITCHLOSS_W = 0.1
ZLOSS_W = 0.0001
BT = 256
N_TOK = 2048


def _moe_body(x_ref, wgt_ref, w1_ref, w2_ref, y_ref, loss_ref,
              gates_ref, gsum_ref, psum_ref, cnt_ref, zsum_ref):
    tb = pl.program_id(0)
    e = pl.program_id(1)
    nt = pl.num_programs(0)

    @pl.when(jnp.logical_and(tb == 0, e == 0))
    def _init():
        gsum_ref[...] = jnp.zeros_like(gsum_ref)
        psum_ref[...] = jnp.zeros_like(psum_ref)
        cnt_ref[...] = jnp.zeros_like(cnt_ref)
        zsum_ref[...] = jnp.zeros_like(zsum_ref)

    @pl.when(e == 0)
    def _gating():
        xb = x_ref[...]
        logits = jnp.dot(xb, wgt_ref[...],
                         preferred_element_type=jnp.float32)
        mx = jnp.max(logits, axis=1, keepdims=True)
        ex = jnp.exp(logits - mx)
        se = jnp.sum(ex, axis=1, keepdims=True)
        probs = ex / se
        m1 = jnp.max(probs, axis=1, keepdims=True)
        lane = jax.lax.broadcasted_iota(jnp.int32, probs.shape, 1)
        first_m1 = jnp.min(jnp.where(probs == m1, lane, NE), axis=1,
                           keepdims=True)
        probs_wo = jnp.where(lane == first_m1, -jnp.inf, probs)
        m2 = jnp.max(probs_wo, axis=1, keepdims=True)
        keep = probs >= m2
        gates = jnp.where(keep, probs, 0.0)
        gates_ref[...] = gates
        gsum_ref[...] += jnp.sum(gates, axis=0, keepdims=True)
        psum_ref[...] += jnp.sum(probs, axis=0, keepdims=True)
        cnt_ref[...] += jnp.sum(
            jnp.where(gates > 0, 1.0, 0.0), axis=0, keepdims=True)
        lse = mx[:, 0] + jnp.log(se[:, 0])
        zsum_ref[...] += jnp.sum(lse * lse).reshape(1, 1)

    h = jnp.maximum(jnp.dot(x_ref[...], w1_ref[0],
                            preferred_element_type=jnp.float32), 0.0)
    o = jnp.dot(h.astype(jnp.bfloat16), w2_ref[0],
                preferred_element_type=jnp.float32)
    lane8 = jax.lax.broadcasted_iota(jnp.int32, (BT, NE), 1)
    ge = jnp.sum(jnp.where(lane8 == e, gates_ref[...], 0.0), axis=1,
                 keepdims=True)

    @pl.when(e == 0)
    def _y_first():
        y_ref[...] = ge * o

    @pl.when(e > 0)
    def _y_acc():
        y_ref[...] += ge * o

    @pl.when(jnp.logical_and(tb == nt - 1, e == NE - 1))
    def _loss():
        gs = gsum_ref[...]
        w = gs / jnp.maximum(jnp.sum(jnp.abs(gs)), 1e-12)
        wm = jnp.mean(w)
        var = jnp.sum((w - wm) ** 2) / (NE - 1)
        cvloss = CVLOSS_W * var / (wm * wm + 1e-10)
        pn = psum_ref[...]
        pn = pn / jnp.maximum(jnp.sum(jnp.abs(pn)), 1e-12)
        cn = cnt_ref[...]
        cn = cn / jnp.maximum(jnp.sum(jnp.abs(cn)), 1e-12)
        switchloss = SWITCHLOSS_W * (1.0 - jnp.sum(pn * cn)) * NE
        zloss = ZLOSS_W * jnp.sum(zsum_ref[...]) / N_TOK
        loss_ref[...] = (cvloss + switchloss + zloss).reshape(1, 1)


@jax.jit
def _moe_fused(xf, wgt, w1, w2):
    nt = N_TOK // BT
    y, loss = pl.pallas_call(
        _moe_body,
        grid=(nt, NE),
        in_specs=[
            pl.BlockSpec((BT, D_IN), lambda tb, e: (tb, 0)),
            pl.BlockSpec((D_IN, NE), lambda tb, e: (0, 0)),
            pl.BlockSpec((1, D_IN, D_HID), lambda tb, e: (e, 0, 0)),
            pl.BlockSpec((1, D_HID, D_IN), lambda tb, e: (e, 0, 0)),
        ],
        out_specs=[
            pl.BlockSpec((BT, D_IN), lambda tb, e: (tb, 0)),
            pl.BlockSpec((1, 1), lambda tb, e: (0, 0)),
        ],
        out_shape=[
            jax.ShapeDtypeStruct((N_TOK, D_IN), jnp.float32),
            jax.ShapeDtypeStruct((1, 1), jnp.float32),
        ],
        scratch_shapes=[
            pltpu.VMEM((BT, NE), jnp.float32),
            pltpu.VMEM((1, NE), jnp.float32),
            pltpu.VMEM((1, NE), jnp.float32),
            pltpu.VMEM((1, NE), jnp.float32),
            pltpu.VMEM((1, 1), jnp.float32),
        ],
        compiler_params=pltpu.CompilerParams(
            dimension_semantics=("arbitrary", "arbitrary"),
        ),
    )(xf.astype(jnp.bfloat16), wgt.astype(jnp.bfloat16),
      w1.astype(jnp.bfloat16), w2.astype(jnp.bfloat16))
    return y, loss


def kernel(x, Wg, W1, W2):
    bsz, length, emb = x.shape
    xf = x.reshape(-1, emb)
    y, loss = _moe_fused(xf, Wg.T, W1, W2)
    return y.reshape(bsz, length, emb), loss[0, 0]


# speedup vs baseline: 1.9322x; 1.4905x over previous
import jax
import jax.numpy as jnp
from jax.experimental import pallas as pl
from jax.experimental.pallas import tpu as pltpu

NE = 8
D_IN = 1024
D_HID = 512
CVLOSS_W = 0.01
SWITCHLOSS_W = 0.1
ZLOSS_W = 0.0001
BT = 256
N_TOK = 2048


def _moe_body(x_ref, wgt_ref, w1_ref, w2_ref, y_ref, loss_ref,
              gsum_ref, psum_ref, cnt_ref, zsum_ref):
    tb = pl.program_id(0)
    nt = pl.num_programs(0)

    @pl.when(tb == 0)
    def _init():
        gsum_ref[...] = jnp.zeros_like(gsum_ref)
        psum_ref[...] = jnp.zeros_like(psum_ref)
        cnt_ref[...] = jnp.zeros_like(cnt_ref)
        zsum_ref[...] = jnp.zeros_like(zsum_ref)

    xb = x_ref[...]
    logits = jnp.dot(xb, wgt_ref[...],
                     preferred_element_type=jnp.float32)
    mx = jnp.max(logits, axis=1, keepdims=True)
    ex = jnp.exp(logits - mx)
    se = jnp.sum(ex, axis=1, keepdims=True)
    probs = ex / se
    m1 = jnp.max(probs, axis=1, keepdims=True)
    lane = jax.lax.broadcasted_iota(jnp.int32, probs.shape, 1)
    first_m1 = jnp.min(jnp.where(probs == m1, lane, NE), axis=1,
                       keepdims=True)
    probs_wo = jnp.where(lane == first_m1, -jnp.inf, probs)
    m2 = jnp.max(probs_wo, axis=1, keepdims=True)
    keep = probs >= m2
    gates = jnp.where(keep, probs, 0.0)
    gsum_ref[...] += jnp.sum(gates, axis=0, keepdims=True)
    psum_ref[...] += jnp.sum(probs, axis=0, keepdims=True)
    cnt_ref[...] += jnp.sum(
        jnp.where(gates > 0, 1.0, 0.0), axis=0, keepdims=True)
    lse = mx[:, 0] + jnp.log(se[:, 0])
    zsum_ref[...] += jnp.sum(lse * lse).reshape(1, 1)

    acc = jnp.zeros((BT, D_IN), jnp.float32)
    for e in range(NE):
        h = jnp.maximum(jnp.dot(xb, w1_ref[e],
                                preferred_element_type=jnp.float32), 0.0)
        o = jnp.dot(h.astype(jnp.bfloat16), w2_ref[e],
                    preferred_element_type=jnp.float32)
        acc = acc + gates[:, e:e + 1] * o
    y_ref[...] = acc

    @pl.when(tb == nt - 1)
    def _loss():
        gs = gsum_ref[...]
        w = gs / jnp.maximum(jnp.sum(jnp.abs(gs)), 1e-12)
        wm = jnp.mean(w)
        var = jnp.sum((w - wm) ** 2) / (NE - 1)
        cvloss = CVLOSS_W * var / (wm * wm + 1e-10)
        pn = psum_ref[...]
        pn = pn / jnp.maximum(jnp.sum(jnp.abs(pn)), 1e-12)
        cn = cnt_ref[...]
        cn = cn / jnp.maximum(jnp.sum(jnp.abs(cn)), 1e-12)
        switchloss = SWITCHLOSS_W * (1.0 - jnp.sum(pn * cn)) * NE
        zloss = ZLOSS_W * jnp.sum(zsum_ref[...]) / N_TOK
        loss_ref[...] = (cvloss + switchloss + zloss).reshape(1, 1)


@jax.jit
def _moe_fused(xf, wgt, w1, w2):
    nt = N_TOK // BT
    y, loss = pl.pallas_call(
        _moe_body,
        grid=(nt,),
        in_specs=[
            pl.BlockSpec((BT, D_IN), lambda tb: (tb, 0)),
            pl.BlockSpec((D_IN, NE), lambda tb: (0, 0)),
            pl.BlockSpec((NE, D_IN, D_HID), lambda tb: (0, 0, 0)),
            pl.BlockSpec((NE, D_HID, D_IN), lambda tb: (0, 0, 0)),
        ],
        out_specs=[
            pl.BlockSpec((BT, D_IN), lambda tb: (tb, 0)),
            pl.BlockSpec((1, 1), lambda tb: (0, 0)),
        ],
        out_shape=[
            jax.ShapeDtypeStruct((N_TOK, D_IN), jnp.float32),
            jax.ShapeDtypeStruct((1, 1), jnp.float32),
        ],
        scratch_shapes=[
            pltpu.VMEM((1, NE), jnp.float32),
            pltpu.VMEM((1, NE), jnp.float32),
            pltpu.VMEM((1, NE), jnp.float32),
            pltpu.VMEM((1, 1), jnp.float32),
        ],
        compiler_params=pltpu.CompilerParams(
            dimension_semantics=("arbitrary",),
        ),
    )(xf.astype(jnp.bfloat16), wgt.astype(jnp.bfloat16),
      w1.astype(jnp.bfloat16), w2.astype(jnp.bfloat16))
    return y, loss


def kernel(x, Wg, W1, W2):
    bsz, length, emb = x.shape
    xf = x.reshape(-1, emb)
    y, loss = _moe_fused(xf, Wg.T, W1, W2)
    return y.reshape(bsz, length, emb), loss[0, 0]


# speedup vs baseline: 2.1196x; 1.0970x over previous
import jax
import jax.numpy as jnp
from jax.experimental import pallas as pl
from jax.experimental.pallas import tpu as pltpu

NE = 8
D_IN = 1024
D_HID = 512
CVLOSS_W = 0.01
SWITCHLOSS_W = 0.1
ZLOSS_W = 0.0001
BT = 512
N_TOK = 2048


def _moe_body(x_ref, wgt_ref, w1_ref, w2_ref, y_ref, loss_ref,
              gsum_ref, psum_ref, cnt_ref, zsum_ref):
    tb = pl.program_id(0)
    nt = pl.num_programs(0)

    @pl.when(tb == 0)
    def _init():
        gsum_ref[...] = jnp.zeros_like(gsum_ref)
        psum_ref[...] = jnp.zeros_like(psum_ref)
        cnt_ref[...] = jnp.zeros_like(cnt_ref)
        zsum_ref[...] = jnp.zeros_like(zsum_ref)

    xb = x_ref[...]
    logits = jnp.dot(xb, wgt_ref[...],
                     preferred_element_type=jnp.float32)
    mx = jnp.max(logits, axis=1, keepdims=True)
    ex = jnp.exp(logits - mx)
    se = jnp.sum(ex, axis=1, keepdims=True)
    probs = ex / se
    m1 = jnp.max(probs, axis=1, keepdims=True)
    lane = jax.lax.broadcasted_iota(jnp.int32, probs.shape, 1)
    first_m1 = jnp.min(jnp.where(probs == m1, lane, NE), axis=1,
                       keepdims=True)
    probs_wo = jnp.where(lane == first_m1, -jnp.inf, probs)
    m2 = jnp.max(probs_wo, axis=1, keepdims=True)
    keep = probs >= m2
    gates = jnp.where(keep, probs, 0.0)
    gsum_ref[...] += jnp.sum(gates, axis=0, keepdims=True)
    psum_ref[...] += jnp.sum(probs, axis=0, keepdims=True)
    cnt_ref[...] += jnp.sum(
        jnp.where(gates > 0, 1.0, 0.0), axis=0, keepdims=True)
    lse = mx[:, 0] + jnp.log(se[:, 0])
    zsum_ref[...] += jnp.sum(lse * lse).reshape(1, 1)

    acc = jnp.zeros((BT, D_IN), jnp.float32)
    for e in range(NE):
        h = jnp.maximum(jnp.dot(xb, w1_ref[e],
                                preferred_element_type=jnp.float32), 0.0)
        o = jnp.dot(h.astype(jnp.bfloat16), w2_ref[e],
                    preferred_element_type=jnp.float32)
        acc = acc + gates[:, e:e + 1] * o
    y_ref[...] = acc

    @pl.when(tb == nt - 1)
    def _loss():
        gs = gsum_ref[...]
        w = gs / jnp.maximum(jnp.sum(jnp.abs(gs)), 1e-12)
        wm = jnp.mean(w)
        var = jnp.sum((w - wm) ** 2) / (NE - 1)
        cvloss = CVLOSS_W * var / (wm * wm + 1e-10)
        pn = psum_ref[...]
        pn = pn / jnp.maximum(jnp.sum(jnp.abs(pn)), 1e-12)
        cn = cnt_ref[...]
        cn = cn / jnp.maximum(jnp.sum(jnp.abs(cn)), 1e-12)
        switchloss = SWITCHLOSS_W * (1.0 - jnp.sum(pn * cn)) * NE
        zloss = ZLOSS_W * jnp.sum(zsum_ref[...]) / N_TOK
        loss_ref[...] = (cvloss + switchloss + zloss).reshape(1, 1)


@jax.jit
def _moe_fused(xf, wgt, w1, w2):
    nt = N_TOK // BT
    y, loss = pl.pallas_call(
        _moe_body,
        grid=(nt,),
        in_specs=[
            pl.BlockSpec((BT, D_IN), lambda tb: (tb, 0)),
            pl.BlockSpec((D_IN, NE), lambda tb: (0, 0)),
            pl.BlockSpec((NE, D_IN, D_HID), lambda tb: (0, 0, 0)),
            pl.BlockSpec((NE, D_HID, D_IN), lambda tb: (0, 0, 0)),
        ],
        out_specs=[
            pl.BlockSpec((BT, D_IN), lambda tb: (tb, 0)),
            pl.BlockSpec((1, 1), lambda tb: (0, 0)),
        ],
        out_shape=[
            jax.ShapeDtypeStruct((N_TOK, D_IN), jnp.float32),
            jax.ShapeDtypeStruct((1, 1), jnp.float32),
        ],
        scratch_shapes=[
            pltpu.VMEM((1, NE), jnp.float32),
            pltpu.VMEM((1, NE), jnp.float32),
            pltpu.VMEM((1, NE), jnp.float32),
            pltpu.VMEM((1, 1), jnp.float32),
        ],
        compiler_params=pltpu.CompilerParams(
            dimension_semantics=("arbitrary",),
        ),
    )(xf.astype(jnp.bfloat16), wgt.astype(jnp.bfloat16),
      w1.astype(jnp.bfloat16), w2.astype(jnp.bfloat16))
    return y, loss


def kernel(x, Wg, W1, W2):
    bsz, length, emb = x.shape
    xf = x.reshape(-1, emb)
    y, loss = _moe_fused(xf, Wg.T, W1, W2)
    return y.reshape(bsz, length, emb), loss[0, 0]
